# baseline (speedup 1.0000x reference)
import functools
import math

import jax
import jax.numpy as jnp
from jax import lax
from jax.experimental import pallas as pl
from jax.experimental.pallas import tpu as pltpu
from jax.experimental.pallas import tpu_sc as plsc

D_MODEL = 64
SCALE = math.sqrt(D_MODEL)


@functools.partial(jax.jit, static_argnames=("n_s", "n_b"))
def _emb_lookup(x_t, lut_pairs, n_s, n_b):
    info = plsc.get_sparse_core_info()
    nc, ns = info.num_cores, info.num_subcores
    nw = nc * ns
    bpw = n_b // nw
    mesh = plsc.VectorSubcoreMesh(core_axis_name="c", subcore_axis_name="s")

    @functools.partial(
        pl.kernel,
        mesh=mesh,
        out_type=jax.ShapeDtypeStruct((n_s, D_MODEL, n_b), jnp.float32),
        scratch_types=[
            pltpu.VMEM((n_s, bpw), jnp.int32),
            pltpu.VMEM((2, bpw), jnp.int32),
            pltpu.VMEM((2 * bpw, 128), jnp.float32),
            pltpu.VMEM((2 * D_MODEL, bpw), jnp.float32),
            pltpu.SemaphoreType.DMA((2,)),
            pltpu.SemaphoreType.DMA((2,)),
        ],
        compiler_params=pltpu.CompilerParams(needs_layout_passes=False),
    )
    def k(xt_hbm, tbl_hbm, out_hbm, xv, pid_v, rows_v, out_v, sem_g, sem_w):
        wid = lax.axis_index("s") * nc + lax.axis_index("c")
        b0 = wid * bpw
        pltpu.sync_copy(xt_hbm.at[:, pl.ds(b0, bpw)], xv)
        iota16 = lax.iota(jnp.int32, 16)

        def compute_pid(s, buf):
            for g in range(bpw // 16):
                idx16 = xv[s, pl.ds(g * 16, 16)]
                pid_v[buf, pl.ds(g * 16, 16)] = idx16 >> 1

        def start_gather(buf):
            pltpu.async_copy(
                tbl_hbm.at[pid_v.at[buf]],
                rows_v.at[pl.ds(buf * bpw, bpw), :],
                sem_g.at[buf],
            )

        def wait_gather(buf):
            pltpu.make_async_copy(
                tbl_hbm.at[pid_v.at[buf]],
                rows_v.at[pl.ds(buf * bpw, bpw), :],
                sem_g.at[buf],
            ).wait()

        def start_write(s, buf):
            pltpu.async_copy(
                out_v.at[pl.ds(buf * D_MODEL, D_MODEL), :],
                out_hbm.at[s, :, pl.ds(b0, bpw)],
                sem_w.at[buf],
            )

        def wait_write(s, buf):
            pltpu.make_async_copy(
                out_v.at[pl.ds(buf * D_MODEL, D_MODEL), :],
                out_hbm.at[s, :, pl.ds(b0, bpw)],
                sem_w.at[buf],
            ).wait()

        def assemble(s, buf):
            for g in range(bpw // 16):
                idx16 = xv[s, pl.ds(g * 16, 16)]
                half = (idx16 & 1) << 6
                rows16 = (buf * bpw + g * 16) + iota16

                for d in range(D_MODEL):
                    col16 = half + d
                    v = plsc.load_gather(rows_v, [rows16, col16])
                    out_v[buf * D_MODEL + d, pl.ds(g * 16, 16)] = v * SCALE

        compute_pid(0, 0)
        start_gather(0)

        def sbody(s, _):
            buf = s % 2
            nbuf = (s + 1) % 2

            @pl.when(s < n_s - 1)
            def _():
                compute_pid(s + 1, nbuf)
                start_gather(nbuf)

            wait_gather(buf)

            @pl.when(s >= 2)
            def _():
                wait_write(s - 2, buf)

            assemble(s, buf)
            start_write(s, buf)
            return 0

        lax.fori_loop(0, n_s, sbody, 0)
        wait_write(n_s - 2, 0)
        wait_write(n_s - 1, 1)

    return k(x_t, lut_pairs)


def kernel(x, lut):
    b, s = x.shape
    v, d = lut.shape
    x_t = jnp.transpose(x).astype(jnp.int32)
    lut_pairs = lut.reshape(v // 2, 2 * d)
    out3 = _emb_lookup(x_t, lut_pairs, s, b)
    return jnp.transpose(out3, (2, 0, 1))


# speedup vs baseline: 1.0000x; 1.0000x over previous
import functools
import math

import jax
import jax.numpy as jnp
from jax import lax
from jax.experimental import pallas as pl
from jax.experimental.pallas import tpu as pltpu
from jax.experimental.pallas import tpu_sc as plsc

D_MODEL = 64
SCALE = math.sqrt(D_MODEL)


@functools.partial(jax.jit, static_argnames=("n_s", "n_b"))
def _emb_lookup(x_t, lut_pairs, n_s, n_b):
    info = plsc.get_sparse_core_info()
    nc, ns = info.num_cores, info.num_subcores
    nw = nc * ns
    bpw = n_b // nw
    mesh = plsc.VectorSubcoreMesh(core_axis_name="c", subcore_axis_name="s")

    @functools.partial(
        pl.kernel,
        mesh=mesh,
        out_type=jax.ShapeDtypeStruct((n_s, D_MODEL, n_b), jnp.float32),
        scratch_types=[
            pltpu.VMEM((n_s, bpw), jnp.int32),
            pltpu.VMEM((2, bpw), jnp.int32),
            pltpu.VMEM((2 * bpw, 128), jnp.float32),
            pltpu.VMEM((2 * D_MODEL, bpw), jnp.float32),
            pltpu.SemaphoreType.DMA((2,)),
            pltpu.SemaphoreType.DMA((2,)),
        ],
        compiler_params=pltpu.CompilerParams(
            needs_layout_passes=False, disable_bounds_checks=True
        ),
    )
    def k(xt_hbm, tbl_hbm, out_hbm, xv, pid_v, rows_v, out_v, sem_g, sem_w):
        wid = lax.axis_index("s") * nc + lax.axis_index("c")
        b0 = wid * bpw
        pltpu.sync_copy(xt_hbm.at[:, pl.ds(b0, bpw)], xv)
        iota16 = lax.iota(jnp.int32, 16)

        def compute_pid(s, buf):
            for g in range(bpw // 16):
                idx16 = xv[s, pl.ds(g * 16, 16)]
                pid_v[buf, pl.ds(g * 16, 16)] = idx16 >> 1

        def start_gather(buf):
            pltpu.async_copy(
                tbl_hbm.at[pid_v.at[buf]],
                rows_v.at[pl.ds(buf * bpw, bpw), :],
                sem_g.at[buf],
            )

        def wait_gather(buf):
            pltpu.make_async_copy(
                tbl_hbm.at[pid_v.at[buf]],
                rows_v.at[pl.ds(buf * bpw, bpw), :],
                sem_g.at[buf],
            ).wait()

        def start_write(s, buf):
            pltpu.async_copy(
                out_v.at[pl.ds(buf * D_MODEL, D_MODEL), :],
                out_hbm.at[s, :, pl.ds(b0, bpw)],
                sem_w.at[buf],
            )

        def wait_write(s, buf):
            pltpu.make_async_copy(
                out_v.at[pl.ds(buf * D_MODEL, D_MODEL), :],
                out_hbm.at[s, :, pl.ds(b0, bpw)],
                sem_w.at[buf],
            ).wait()

        def assemble(s, buf):
            for g in range(bpw // 16):
                idx16 = xv[s, pl.ds(g * 16, 16)]
                half = (idx16 & 1) << 6
                rows16 = (buf * bpw + g * 16) + iota16

                for d in range(D_MODEL):
                    col16 = half + d
                    v = plsc.load_gather(rows_v, [rows16, col16])
                    out_v[buf * D_MODEL + d, pl.ds(g * 16, 16)] = v * SCALE

        compute_pid(0, 0)
        start_gather(0)

        def sbody(s, _):
            buf = s % 2
            nbuf = (s + 1) % 2

            @pl.when(s < n_s - 1)
            def _():
                compute_pid(s + 1, nbuf)
                start_gather(nbuf)

            wait_gather(buf)

            @pl.when(s >= 2)
            def _():
                wait_write(s - 2, buf)

            assemble(s, buf)
            start_write(s, buf)
            return 0

        lax.fori_loop(0, n_s, sbody, 0)
        wait_write(n_s - 2, 0)
        wait_write(n_s - 1, 1)

    return k(x_t, lut_pairs)


def kernel(x, lut):
    b, s = x.shape
    v, d = lut.shape
    x_t = jnp.transpose(x).astype(jnp.int32)
    lut_pairs = lut.reshape(v // 2, 2 * d)
    out3 = _emb_lookup(x_t, lut_pairs, s, b)
    return jnp.transpose(out3, (2, 0, 1))


# speedup vs baseline: 1.5634x; 1.5634x over previous
import functools
import math

import jax
import jax.numpy as jnp
from jax import lax
from jax.experimental import pallas as pl
from jax.experimental.pallas import tpu as pltpu
from jax.experimental.pallas import tpu_sc as plsc

D_MODEL = 64
SCALE = math.sqrt(D_MODEL)


@functools.partial(jax.jit, static_argnames=("n_s", "n_b"))
def _emb_lookup(x_t, lut_pairs, n_s, n_b):
    info = plsc.get_sparse_core_info()
    nc, ns = info.num_cores, info.num_subcores
    nw = nc * ns
    bpw = n_b // nw
    mesh = plsc.VectorSubcoreMesh(core_axis_name="c", subcore_axis_name="s")

    @functools.partial(
        pl.kernel,
        mesh=mesh,
        out_type=jax.ShapeDtypeStruct((n_s, D_MODEL, n_b), jnp.float32),
        scratch_types=[
            pltpu.VMEM((n_s, bpw), jnp.int32),
            pltpu.VMEM((2, bpw), jnp.int32),
            pltpu.VMEM((2 * bpw, 128), jnp.float32),
            pltpu.VMEM((2 * D_MODEL, bpw), jnp.float32),
            pltpu.SemaphoreType.DMA((2,)),
            pltpu.SemaphoreType.DMA((2,)),
        ],
        compiler_params=pltpu.CompilerParams(
            needs_layout_passes=False, disable_bounds_checks=True
        ),
    )
    def k(xt_hbm, tbl_hbm, out_hbm, xv, pid_v, rows_v, out_v, sem_g, sem_w):
        wid = lax.axis_index("s") * nc + lax.axis_index("c")
        b0 = wid * bpw
        pltpu.sync_copy(xt_hbm.at[:, pl.ds(b0, bpw)], xv)
        iota16 = lax.iota(jnp.int32, 16)

        def compute_pid(s, buf):
            for g in range(bpw // 16):
                idx16 = xv[s, pl.ds(g * 16, 16)]
                pid_v[buf, pl.ds(g * 16, 16)] = idx16 >> 1

        def start_gather(buf):
            pltpu.async_copy(
                tbl_hbm.at[pid_v.at[buf]],
                rows_v.at[pl.ds(buf * bpw, bpw), :],
                sem_g.at[buf],
            )

        def wait_gather(buf):
            pltpu.make_async_copy(
                tbl_hbm.at[pid_v.at[buf]],
                rows_v.at[pl.ds(buf * bpw, bpw), :],
                sem_g.at[buf],
            ).wait()

        def start_write(s, buf):
            pltpu.async_copy(
                out_v.at[pl.ds(buf * D_MODEL, D_MODEL), :],
                out_hbm.at[s, :, pl.ds(b0, bpw)],
                sem_w.at[buf],
            )

        def wait_write(s, buf):
            pltpu.make_async_copy(
                out_v.at[pl.ds(buf * D_MODEL, D_MODEL), :],
                out_hbm.at[s, :, pl.ds(b0, bpw)],
                sem_w.at[buf],
            ).wait()

        def assemble(s, buf):
            for g in range(bpw // 16):
                idx16 = xv[s, pl.ds(g * 16, 16)]
                half = (idx16 & 1) << 6
                rows16 = (buf * bpw + g * 16) + iota16

                @plsc.parallel_loop(0, D_MODEL, unroll=8)
                def _dbody(d):
                    col16 = half + d
                    v = plsc.load_gather(rows_v, [rows16, col16])
                    out_v[buf * D_MODEL + d, pl.ds(g * 16, 16)] = v * SCALE

        compute_pid(0, 0)
        start_gather(0)

        def sbody(s, _):
            buf = s % 2
            nbuf = (s + 1) % 2

            @pl.when(s < n_s - 1)
            def _():
                compute_pid(s + 1, nbuf)
                start_gather(nbuf)

            wait_gather(buf)

            @pl.when(s >= 2)
            def _():
                wait_write(s - 2, buf)

            assemble(s, buf)
            start_write(s, buf)
            return 0

        lax.fori_loop(0, n_s, sbody, 0)
        wait_write(n_s - 2, 0)
        wait_write(n_s - 1, 1)

    return k(x_t, lut_pairs)


def kernel(x, lut):
    b, s = x.shape
    v, d = lut.shape
    x_t = jnp.transpose(x).astype(jnp.int32)
    lut_pairs = lut.reshape(v // 2, 2 * d)
    out3 = _emb_lookup(x_t, lut_pairs, s, b)
    return jnp.transpose(out3, (2, 0, 1))
